# baseline (speedup 1.0000x reference)
import math

import jax
import jax.numpy as jnp
from jax.experimental import pallas as pl
from jax.experimental.pallas import tpu as pltpu

H = 256
W = 256
HW = H * W
N_RAW = 10000
CHUNK = 128
NPAD = 10240
NCHUNKS = NPAD // CHUNK
HALF = 128
NHALF = W // HALF
ROWGRP = 8
NGRP = H // ROWGRP
PIX_STEP = 2.0 / H
GY0 = -1.0 + 0.5 * PIX_STEP
LOG2E = math.log2(math.e)


def _prep_kernel(xyz_ref, sc_ref, rot_ref, fdc_ref, nf_ref, op_ref, gx_ref,
                 w0_ref, m0_ref, rho_ref, amp_ref):
    xy = jnp.tanh(xyz_ref[...])
    scaling = jnp.abs(sc_ref[...] + 0.5)
    theta = jax.nn.sigmoid(rot_ref[...]) * (2.0 * math.pi)
    normf = jnp.exp(nf_ref[...])
    amp = fdc_ref[...] * op_ref[...] * (normf[0:1] * normf[1:2])
    amp_ref[...] = amp.astype(jnp.bfloat16)
    c = jnp.cos(theta)
    s = jnp.sin(theta)
    sx2 = scaling[0:1] ** 2 + 1e-8
    sy2 = scaling[1:2] ** 2 + 1e-8
    covA = c * c * sx2 + s * s * sy2
    covB = c * s * (sx2 - sy2)
    covC = s * s * sx2 + c * c * sy2
    det = covA * covC - covB * covB + 1e-12
    Ah = (-0.5 * LOG2E) * covC / det
    Bh = LOG2E * covB / det
    Ch = (-0.5 * LOG2E) * covA / det

    gx = gx_ref[...]
    k = PIX_STEP
    for cidx in range(NCHUNKS):
        cs = slice(cidx * CHUNK, (cidx + 1) * CHUNK)
        pxc = xy[0:1, cs].reshape(CHUNK, 1)
        pyc = xy[1:2, cs].reshape(CHUNK, 1)
        Ac = Ah[:, cs].reshape(CHUNK, 1)
        Bc = Bh[:, cs].reshape(CHUNK, 1)
        Cc = Ch[:, cs].reshape(CHUNK, 1)
        dx = gx - pxc
        u0 = (Ac * dx - Bc * pyc) * dx + Cc * (pyc * pyc)
        u1 = Bc * dx - (2.0 * Cc * pyc)
        u2 = Cc + 0.0 * dx
        d0 = (k * u1 + (k * k) * u2) + (2.0 * k * GY0) * u2
        w0_ref[cs, :] = jnp.exp2(u0 + GY0 * (u1 + GY0 * u2))
        m0_ref[cs, :] = jnp.exp2(d0)
        rho_ref[cs, :] = jnp.exp2((2.0 * k * k) * u2)


def _raster_kernel(w0_ref, m0_ref, rho_ref, amp_ref, out_ref):
    j = pl.program_id(0)
    ampb = amp_ref[...]

    @pl.when(j == 0)
    def _init():
        out_ref[...] = jnp.zeros((3, HW), jnp.float32)

    for h in range(NHALF):
        hs = slice(h * HALF, (h + 1) * HALF)
        w = w0_ref[:, hs]
        m = m0_ref[:, hs]
        rh = rho_ref[:, hs]
        for g in range(NGRP):
            rows = []
            for r in range(ROWGRP):
                rows.append(w.astype(jnp.bfloat16))
                if g * ROWGRP + r + 1 < H:
                    w = w * m
                    m = m * rh
            wcat = jnp.concatenate(rows, axis=1)
            contrib = jax.lax.dot_general(
                ampb, wcat, (((1,), (0,)), ((), ())),
                preferred_element_type=jnp.float32)
            col = h * (H * HALF) + g * (ROWGRP * HALF)
            out_ref[:, col:col + ROWGRP * HALF] += contrib

    @pl.when(j == NCHUNKS - 1)
    def _finish():
        out_ref[...] = jnp.clip(out_ref[...], 0.0, 1.0)


def kernel(_xyz, _scaling, _rotation, _features_dc, _normf, _opacity):
    f32 = jnp.float32
    pad = ((0, NPAD - N_RAW), (0, 0))
    xyzT = jnp.pad(_xyz.astype(f32), pad).T
    scT = jnp.pad(_scaling.astype(f32), pad).T
    rotT = jnp.pad(_rotation.astype(f32), pad).T
    fdcT = jnp.pad(_features_dc.astype(f32), pad).T
    nfT = jnp.pad(_normf.astype(f32), pad).T
    opT = jnp.pad(_opacity.astype(f32), pad).T

    gx = ((jnp.arange(W, dtype=f32) + 0.5) / W * 2.0 - 1.0).reshape(1, W)

    full = lambda shape: pl.BlockSpec(shape, lambda: (0,) * len(shape))
    w0, m0, rho, ampb = pl.pallas_call(
        _prep_kernel,
        grid=(),
        in_specs=[full((2, NPAD)), full((2, NPAD)), full((1, NPAD)),
                  full((3, NPAD)), full((2, NPAD)), full((1, NPAD)),
                  full((1, W))],
        out_specs=[full((NPAD, W)), full((NPAD, W)), full((NPAD, W)),
                   full((3, NPAD))],
        out_shape=[jax.ShapeDtypeStruct((NPAD, W), f32),
                   jax.ShapeDtypeStruct((NPAD, W), f32),
                   jax.ShapeDtypeStruct((NPAD, W), f32),
                   jax.ShapeDtypeStruct((3, NPAD), jnp.bfloat16)],
    )(xyzT, scT, rotT, fdcT, nfT, opT, gx)

    fld_spec = pl.BlockSpec((CHUNK, W), lambda j: (j, 0))
    out = pl.pallas_call(
        _raster_kernel,
        grid=(NCHUNKS,),
        in_specs=[fld_spec, fld_spec, fld_spec,
                  pl.BlockSpec((3, CHUNK), lambda j: (0, j))],
        out_specs=pl.BlockSpec((3, HW), lambda j: (0, 0)),
        out_shape=jax.ShapeDtypeStruct((3, HW), f32),
        compiler_params=pltpu.CompilerParams(
            dimension_semantics=("arbitrary",),
        ),
    )(w0, m0, rho, ampb)

    img = out.reshape(3, NHALF, H, HALF).transpose(0, 2, 1, 3)
    return img.reshape(1, 3, H, W)


# speedup vs baseline: 1.2686x; 1.2686x over previous
import math

import jax
import jax.numpy as jnp
from jax.experimental import pallas as pl
from jax.experimental.pallas import tpu as pltpu

H = 256
W = 256
HW = H * W
N_RAW = 10000
CHUNK = 128
NPAD = 10240
NCHUNKS = NPAD // CHUNK
HALF = 128
NHALF = W // HALF
ROWGRP = 8
NGRP = H // ROWGRP
PIX_STEP = 2.0 / H
GY0 = -1.0 + 0.5 * PIX_STEP
LOG2E = math.log2(math.e)


def _prep_kernel(xyz_ref, sc_ref, rot_ref, fdc_ref, nf_ref, op_ref, gx_ref,
                 w0_ref, m0_ref, rho_ref, amp_ref):
    xy = jnp.tanh(xyz_ref[...])
    scaling = jnp.abs(sc_ref[...] + 0.5)
    theta = jax.nn.sigmoid(rot_ref[...]) * (2.0 * math.pi)
    normf = jnp.exp(nf_ref[...])
    amp = fdc_ref[...] * op_ref[...] * (normf[0:1] * normf[1:2])
    amp_ref[...] = amp.astype(jnp.bfloat16)
    c = jnp.cos(theta)
    s = jnp.sin(theta)
    sx2 = scaling[0:1] ** 2 + 1e-8
    sy2 = scaling[1:2] ** 2 + 1e-8
    covA = c * c * sx2 + s * s * sy2
    covB = c * s * (sx2 - sy2)
    covC = s * s * sx2 + c * c * sy2
    det = covA * covC - covB * covB + 1e-12
    Ah = (-0.5 * LOG2E) * covC / det
    Bh = LOG2E * covB / det
    Ch = (-0.5 * LOG2E) * covA / det

    gx = gx_ref[...]
    k = PIX_STEP
    for cidx in range(NCHUNKS):
        cs = slice(cidx * CHUNK, (cidx + 1) * CHUNK)
        pxc = xy[0:1, cs].reshape(CHUNK, 1)
        pyc = xy[1:2, cs].reshape(CHUNK, 1)
        Ac = Ah[:, cs].reshape(CHUNK, 1)
        Bc = Bh[:, cs].reshape(CHUNK, 1)
        Cc = Ch[:, cs].reshape(CHUNK, 1)
        dx = gx - pxc
        u0 = (Ac * dx - Bc * pyc) * dx + Cc * (pyc * pyc)
        u1 = Bc * dx - (2.0 * Cc * pyc)
        u2 = Cc + 0.0 * dx
        d0 = (k * u1 + (k * k) * u2) + (2.0 * k * GY0) * u2
        dd = (2.0 * k * k) * u2
        w0_ref[cs, :] = jnp.exp2(u0 + GY0 * (u1 + GY0 * u2))
        m0_ref[cs, :] = jnp.exp2(d0 + 3.5 * dd)
        rho_ref[cs, :] = jnp.exp2(8.0 * dd)


def _raster_kernel(w0_ref, m0_ref, rho_ref, amp_ref, out_ref):
    j = pl.program_id(0)
    ampb = amp_ref[...]

    @pl.when(j == 0)
    def _init():
        out_ref[...] = jnp.zeros((3, HW), jnp.float32)

    for h in range(NHALF):
        hs = slice(h * HALF, (h + 1) * HALF)
        w = w0_ref[:, hs]
        mt = m0_ref[:, hs]
        r8 = rho_ref[:, hs]
        for g in range(NGRP):
            rows = []
            for r in range(ROWGRP):
                rows.append(w.astype(jnp.bfloat16))
                if g * ROWGRP + r + 1 < H:
                    w = w * mt
            if g + 1 < NGRP:
                mt = mt * r8
            wcat = jnp.concatenate(rows, axis=1)
            contrib = jax.lax.dot_general(
                ampb, wcat, (((1,), (0,)), ((), ())),
                preferred_element_type=jnp.float32)
            col = h * (H * HALF) + g * (ROWGRP * HALF)
            out_ref[:, col:col + ROWGRP * HALF] += contrib

    @pl.when(j == NCHUNKS - 1)
    def _finish():
        out_ref[...] = jnp.clip(out_ref[...], 0.0, 1.0)


def kernel(_xyz, _scaling, _rotation, _features_dc, _normf, _opacity):
    f32 = jnp.float32
    pad = ((0, NPAD - N_RAW), (0, 0))
    xyzT = jnp.pad(_xyz.astype(f32), pad).T
    scT = jnp.pad(_scaling.astype(f32), pad).T
    rotT = jnp.pad(_rotation.astype(f32), pad).T
    fdcT = jnp.pad(_features_dc.astype(f32), pad).T
    nfT = jnp.pad(_normf.astype(f32), pad).T
    opT = jnp.pad(_opacity.astype(f32), pad).T

    gx = ((jnp.arange(W, dtype=f32) + 0.5) / W * 2.0 - 1.0).reshape(1, W)

    full = lambda shape: pl.BlockSpec(shape, lambda: (0,) * len(shape))
    w0, m0, rho, ampb = pl.pallas_call(
        _prep_kernel,
        grid=(),
        in_specs=[full((2, NPAD)), full((2, NPAD)), full((1, NPAD)),
                  full((3, NPAD)), full((2, NPAD)), full((1, NPAD)),
                  full((1, W))],
        out_specs=[full((NPAD, W)), full((NPAD, W)), full((NPAD, W)),
                   full((3, NPAD))],
        out_shape=[jax.ShapeDtypeStruct((NPAD, W), f32),
                   jax.ShapeDtypeStruct((NPAD, W), f32),
                   jax.ShapeDtypeStruct((NPAD, W), f32),
                   jax.ShapeDtypeStruct((3, NPAD), jnp.bfloat16)],
    )(xyzT, scT, rotT, fdcT, nfT, opT, gx)

    fld_spec = pl.BlockSpec((CHUNK, W), lambda j: (j, 0))
    out = pl.pallas_call(
        _raster_kernel,
        grid=(NCHUNKS,),
        in_specs=[fld_spec, fld_spec, fld_spec,
                  pl.BlockSpec((3, CHUNK), lambda j: (0, j))],
        out_specs=pl.BlockSpec((3, HW), lambda j: (0, 0)),
        out_shape=jax.ShapeDtypeStruct((3, HW), f32),
        compiler_params=pltpu.CompilerParams(
            dimension_semantics=("arbitrary",),
        ),
    )(w0, m0, rho, ampb)

    img = out.reshape(3, NHALF, H, HALF).transpose(0, 2, 1, 3)
    return img.reshape(1, 3, H, W)


# speedup vs baseline: 1.4146x; 1.1151x over previous
import math

import jax
import jax.numpy as jnp
from jax.experimental import pallas as pl
from jax.experimental.pallas import tpu as pltpu

H = 256
W = 256
HW = H * W
N_RAW = 10000
CHUNK = 128
NPAD = 10240
NCHUNKS = NPAD // CHUNK
HALF = 128
NHALF = W // HALF
ROWGRP = 8
NGRP = H // ROWGRP
PIX_STEP = 2.0 / H
GY0 = -1.0 + 0.5 * PIX_STEP
LOG2E = math.log2(math.e)


F8MAX = 448.0
LOG2_F8MAX = math.log2(F8MAX)


def _prep_kernel(xyz_ref, sc_ref, rot_ref, fdc_ref, nf_ref, op_ref, gx_ref,
                 w0_ref, m0_ref, rho_ref, amp_ref, scale_ref):
    xy = jnp.tanh(xyz_ref[...])
    scaling = jnp.abs(sc_ref[...] + 0.5)
    theta = jax.nn.sigmoid(rot_ref[...]) * (2.0 * math.pi)
    normf = jnp.exp(nf_ref[...])
    amp = fdc_ref[...] * op_ref[...] * (normf[0:1] * normf[1:2])
    ampmax = jnp.minimum(jnp.max(amp, axis=(0, 1), keepdims=True), 1e30)
    s_amp = F8MAX / jnp.maximum(ampmax, 1e-30)
    amp_ref[...] = (amp * s_amp).astype(jnp.float8_e4m3fn)
    scale_ref[...] = ampmax / (F8MAX * F8MAX)
    c = jnp.cos(theta)
    s = jnp.sin(theta)
    sx2 = scaling[0:1] ** 2 + 1e-8
    sy2 = scaling[1:2] ** 2 + 1e-8
    covA = c * c * sx2 + s * s * sy2
    covB = c * s * (sx2 - sy2)
    covC = s * s * sx2 + c * c * sy2
    det = covA * covC - covB * covB + 1e-12
    Ah = (-0.5 * LOG2E) * covC / det
    Bh = LOG2E * covB / det
    Ch = (-0.5 * LOG2E) * covA / det

    gx = gx_ref[...]
    k = PIX_STEP
    for cidx in range(NCHUNKS):
        cs = slice(cidx * CHUNK, (cidx + 1) * CHUNK)
        pxc = xy[0:1, cs].reshape(CHUNK, 1)
        pyc = xy[1:2, cs].reshape(CHUNK, 1)
        Ac = Ah[:, cs].reshape(CHUNK, 1)
        Bc = Bh[:, cs].reshape(CHUNK, 1)
        Cc = Ch[:, cs].reshape(CHUNK, 1)
        dx = gx - pxc
        u0 = (Ac * dx - Bc * pyc) * dx + Cc * (pyc * pyc)
        u1 = Bc * dx - (2.0 * Cc * pyc)
        u2 = Cc + 0.0 * dx
        d0 = (k * u1 + (k * k) * u2) + (2.0 * k * GY0) * u2
        dd = (2.0 * k * k) * u2
        w0_ref[cs, :] = jnp.exp2(u0 + GY0 * (u1 + GY0 * u2) + LOG2_F8MAX)
        m0_ref[cs, :] = jnp.exp2(d0 + 3.5 * dd)
        rho_ref[cs, :] = jnp.exp2(8.0 * dd)


def _raster_kernel(w0_ref, m0_ref, rho_ref, amp_ref, out_ref):
    j = pl.program_id(0)
    ampb = amp_ref[...]

    @pl.when(j == 0)
    def _init():
        out_ref[...] = jnp.zeros((3, HW), jnp.float32)

    for h in range(NHALF):
        hs = slice(h * HALF, (h + 1) * HALF)
        w = w0_ref[:, hs]
        mt = m0_ref[:, hs]
        r8 = rho_ref[:, hs]
        for g in range(NGRP):
            mb = mt.astype(jnp.bfloat16)
            mt2 = mt * mt
            rows = []
            for rr in range(ROWGRP // 2):
                we = w.astype(jnp.bfloat16)
                rows.append(we.astype(jnp.float8_e4m3fn))
                rows.append((we * mb).astype(jnp.float8_e4m3fn))
                if g * ROWGRP + 2 * rr + 2 < H:
                    w = w * mt2
            if g + 1 < NGRP:
                mt = mt * r8
            wcat = jnp.concatenate(rows, axis=1)
            contrib = jax.lax.dot_general(
                ampb, wcat, (((1,), (0,)), ((), ())),
                preferred_element_type=jnp.float32)
            col = h * (H * HALF) + g * (ROWGRP * HALF)
            out_ref[:, col:col + ROWGRP * HALF] += contrib


def kernel(_xyz, _scaling, _rotation, _features_dc, _normf, _opacity):
    f32 = jnp.float32
    pad = ((0, NPAD - N_RAW), (0, 0))
    xyzT = jnp.pad(_xyz.astype(f32), pad).T
    scT = jnp.pad(_scaling.astype(f32), pad).T
    rotT = jnp.pad(_rotation.astype(f32), pad).T
    fdcT = jnp.pad(_features_dc.astype(f32), pad).T
    nfT = jnp.pad(_normf.astype(f32), pad).T
    opT = jnp.pad(_opacity.astype(f32), pad).T

    gx = ((jnp.arange(W, dtype=f32) + 0.5) / W * 2.0 - 1.0).reshape(1, W)

    full = lambda shape: pl.BlockSpec(shape, lambda: (0,) * len(shape))
    w0, m0, rho, ampb, scl = pl.pallas_call(
        _prep_kernel,
        grid=(),
        in_specs=[full((2, NPAD)), full((2, NPAD)), full((1, NPAD)),
                  full((3, NPAD)), full((2, NPAD)), full((1, NPAD)),
                  full((1, W))],
        out_specs=[full((NPAD, W)), full((NPAD, W)), full((NPAD, W)),
                   full((3, NPAD)), full((1, 1))],
        out_shape=[jax.ShapeDtypeStruct((NPAD, W), f32),
                   jax.ShapeDtypeStruct((NPAD, W), f32),
                   jax.ShapeDtypeStruct((NPAD, W), f32),
                   jax.ShapeDtypeStruct((3, NPAD), jnp.float8_e4m3fn),
                   jax.ShapeDtypeStruct((1, 1), f32)],
    )(xyzT, scT, rotT, fdcT, nfT, opT, gx)

    fld_spec = pl.BlockSpec((CHUNK, W), lambda j: (j, 0))
    out = pl.pallas_call(
        _raster_kernel,
        grid=(NCHUNKS,),
        in_specs=[fld_spec, fld_spec, fld_spec,
                  pl.BlockSpec((3, CHUNK), lambda j: (0, j))],
        out_specs=pl.BlockSpec((3, HW), lambda j: (0, 0)),
        out_shape=jax.ShapeDtypeStruct((3, HW), f32),
        compiler_params=pltpu.CompilerParams(
            dimension_semantics=("arbitrary",),
        ),
    )(w0, m0, rho, ampb)

    img = jnp.clip(out * scl[0, 0], 0.0, 1.0)
    img = img.reshape(3, NHALF, H, HALF).transpose(0, 2, 1, 3)
    return img.reshape(1, 3, H, W)
